# Initial kernel scaffold; baseline (speedup 1.0000x reference)
import jax
import jax.numpy as jnp
from jax.experimental import pallas as pl


def kernel(x, edge_index, W_l, W_r, b_l):
    raise NotImplementedError("write your pallas kernel here")


# speedup vs baseline: 4.5954x; 4.5954x over previous
import functools

import jax
import jax.numpy as jnp
from jax import lax
from jax.experimental import pallas as pl
from jax.experimental.pallas import tpu as pltpu, tpu_sc as plsc

NEG_SLOPE = 0.01

NC = 2
NS = 16
L = 16

CH = 80


def _fill_1d(ref, n, value):
    def body(i, _):
        ref[pl.ds(i * L, L)] = jnp.full((L,), value, jnp.float32)
        return 0

    lax.fori_loop(0, n // L, body, 0)


def _fill_2d(ref, rows, value):
    cols = ref.shape[1]

    def body(r, _):
        for j in range(cols // L):
            ref[r, pl.ds(j * L, L)] = jnp.full((L,), value, jnp.float32)
        return 0

    lax.fori_loop(0, rows, body, 0)


def _sc_aggregate(x, src, dst):
    N, D = x.shape
    E = src.shape[0]
    NW = NC * NS
    assert E % NW == 0
    ept = E // NW
    nfull = ept // CH
    assert ept % CH == 0 and CH % 8 == 0
    rpt = (N // NS) & ~7
    rextra = N - NS * rpt
    assert rextra % 8 == 0 and rextra <= rpt
    zrows = CH
    npt = -(-N // (NS * D)) * D
    CR = NS * npt // D
    DCH = 2000
    assert (ept * NS) % DCH == 0 and ept % DCH == 0 and DCH % L == 0
    assert npt % D == 0 and CR % 8 == 0 and NS * npt >= N

    mesh = plsc.VectorSubcoreMesh(core_axis_name="c", subcore_axis_name="s")

    @functools.partial(
        pl.kernel,
        mesh=mesh,
        compiler_params=pltpu.CompilerParams(needs_layout_passes=False),
        out_type=[
            jax.ShapeDtypeStruct((NC, N, D), jnp.float32),
            jax.ShapeDtypeStruct((NC, CR, D), jnp.float32),
        ],
        scratch_types=[
            pltpu.VMEM_SHARED((N, D), jnp.float32),
            pltpu.VMEM_SHARED((CR, D), jnp.float32),
            pltpu.VMEM((CH,), jnp.int32),
            pltpu.VMEM((CH,), jnp.int32),
            pltpu.VMEM((CH, D), jnp.float32),
            pltpu.VMEM((L * npt,), jnp.float32),
            pltpu.VMEM((DCH,), jnp.int32),
            pltpu.VMEM((npt // D, D), jnp.float32),
            pltpu.SemaphoreType.DMA,
        ],
    )
    def sc_kernel(x_hbm, src_hbm, dst_hbm, agg_out, cnt_out,
                  acc, cgrid, sidx, didx, rowbuf, hist, dchunk, ctot, sem):
        c = lax.axis_index("c")
        s = lax.axis_index("s")
        wid = s * NC + c
        iota16 = lax.iota(jnp.int32, 16)
        ones16 = jnp.full((L,), 1.0, jnp.float32)

        zbuf = rowbuf
        _fill_2d(zbuf, zrows, 0.0)
        _fill_1d(hist, L * npt, 0.0)

        row0 = s * rpt

        def zero_rows(tgt, start, count):
            done = 0
            while done < count:
                m = min(zrows, count - done)
                pltpu.sync_copy(zbuf.at[pl.ds(0, m)],
                                tgt.at[pl.ds(start + done, m)])
                done += m

        zero_rows(acc, row0, rpt)
        if rextra:
            @pl.when(s == NS - 1)
            def _():
                zero_rows(acc, NS * rpt, rextra)
        plsc.subcore_barrier()

        ebase = wid * ept

        def chunk(i, _):
            base = ebase + i * CH
            pltpu.sync_copy(src_hbm.at[pl.ds(base, CH)], sidx)
            pltpu.sync_copy(dst_hbm.at[pl.ds(base, CH)], didx)
            pltpu.async_copy(x_hbm.at[sidx], rowbuf, sem).wait()
            pltpu.sync_copy(rowbuf, acc.at[didx], add=True)
            return 0

        lax.fori_loop(0, nfull, chunk, 0)

        nbase = s * npt
        ecore = ept * NS

        def cchunk(q, _):
            e0 = q * DCH
            blk = e0 // ept
            base = (blk * NC + c) * ept + (e0 - blk * ept)
            pltpu.sync_copy(dst_hbm.at[pl.ds(base, DCH)], dchunk)

            def body(k, _):
                dv = dchunk[pl.ds(k * L, L)] - nbase
                msk = (dv >= 0) & (dv < npt)
                plsc.addupdate_scatter(hist, [iota16 * npt + dv], ones16, mask=msk)
                return 0

            lax.fori_loop(0, DCH // L, body, 0)
            return 0

        lax.fori_loop(0, ecore // DCH, cchunk, 0)

        for j in range(npt // L):
            tot = hist[pl.ds(j * L, L)]
            for r in range(1, L):
                tot = tot + hist[pl.ds(r * npt + j * L, L)]
            ctot[(j * L) // D, pl.ds((j * L) % D, L)] = tot
        pltpu.sync_copy(ctot, cgrid.at[pl.ds(s * (npt // D), npt // D)])

        plsc.subcore_barrier()

        pltpu.sync_copy(acc.at[pl.ds(row0, rpt)],
                        agg_out.at[c, pl.ds(row0, rpt)])
        if rextra:
            @pl.when(s == NS - 1)
            def _():
                pltpu.sync_copy(acc.at[pl.ds(NS * rpt, rextra)],
                                agg_out.at[c, pl.ds(NS * rpt, rextra)])

        @pl.when(s == 0)
        def _():
            pltpu.sync_copy(cgrid, cnt_out.at[c])

    return sc_kernel(x, src, dst)


def _tc_body(x_ref, wl_ref, wr_ref, bl_ref, agg_ref, cnt_ref, out_ref):
    a = agg_ref[0] + agg_ref[1]
    cvec = cnt_ref[0] + cnt_ref[1]
    mean = a / jnp.maximum(cvec, 1.0)
    h = (jnp.dot(mean, wl_ref[...], preferred_element_type=jnp.float32)
         + jnp.dot(x_ref[...], wr_ref[...], preferred_element_type=jnp.float32)
         + bl_ref[...])
    h = jnp.where(h >= 0, h, NEG_SLOPE * h)
    nrm = jnp.sqrt(jnp.sum(h * h, axis=-1, keepdims=True))
    out_ref[...] = h / jnp.maximum(nrm, 1e-12)


def _tc_post(x, W_l, W_r, b_l2, agg, cnt):
    N, D = x.shape
    H = W_l.shape[1]
    R = 1000
    assert N % R == 0
    grid = (N // R,)
    return pl.pallas_call(
        _tc_body,
        grid=grid,
        in_specs=[
            pl.BlockSpec((R, D), lambda i: (i, 0)),
            pl.BlockSpec((D, H), lambda i: (0, 0)),
            pl.BlockSpec((D, H), lambda i: (0, 0)),
            pl.BlockSpec((1, H), lambda i: (0, 0)),
            pl.BlockSpec((NC, R, D), lambda i: (0, i, 0)),
            pl.BlockSpec((NC, R, 1), lambda i: (0, i, 0)),
        ],
        out_specs=pl.BlockSpec((R, H), lambda i: (i, 0)),
        out_shape=jax.ShapeDtypeStruct((N, H), jnp.float32),
    )(x, W_l, W_r, b_l2, agg, cnt)


def kernel(x, edge_index, W_l, W_r, b_l):
    src = edge_index[0]
    dst = edge_index[1]
    agg, cnt_grid = _sc_aggregate(x, src, dst)
    N = x.shape[0]
    cnt = cnt_grid.reshape(NC, -1)[:, :N, None]
    return _tc_post(x, W_l, W_r, b_l.reshape(1, -1), agg, cnt)


# speedup vs baseline: 6.7909x; 1.4778x over previous
import functools

import jax
import jax.numpy as jnp
from jax import lax
from jax.experimental import pallas as pl
from jax.experimental.pallas import tpu as pltpu, tpu_sc as plsc

NEG_SLOPE = 0.01

NC = 2
NS = 16
L = 16

CH = 80


def _fill_1d(ref, n, value):
    def body(i, _):
        ref[pl.ds(i * L, L)] = jnp.full((L,), value, jnp.float32)
        return 0

    lax.fori_loop(0, n // L, body, 0)


def _fill_2d(ref, rows, value):
    cols = ref.shape[1]

    def body(r, _):
        for j in range(cols // L):
            ref[r, pl.ds(j * L, L)] = jnp.full((L,), value, jnp.float32)
        return 0

    lax.fori_loop(0, rows, body, 0)


def _sc_aggregate(x, src, dst):
    N, D = x.shape
    E = src.shape[0]
    NW = NC * NS
    assert E % NW == 0
    ept = E // NW
    nfull = ept // CH
    assert ept % CH == 0 and CH % 8 == 0
    rpt = (N // NS) & ~7
    rextra = N - NS * rpt
    assert rextra % 8 == 0 and rextra <= rpt
    zrows = CH
    npt = -(-N // (NS * D)) * D
    CR = NS * npt // D
    DCH = 2000
    assert (ept * NS) % DCH == 0 and ept % DCH == 0 and DCH % L == 0
    assert npt % D == 0 and CR % 8 == 0 and NS * npt >= N

    mesh = plsc.VectorSubcoreMesh(core_axis_name="c", subcore_axis_name="s")

    @functools.partial(
        pl.kernel,
        mesh=mesh,
        compiler_params=pltpu.CompilerParams(needs_layout_passes=False),
        out_type=[
            jax.ShapeDtypeStruct((NC, N, D), jnp.float32),
            jax.ShapeDtypeStruct((NC, CR, D), jnp.float32),
        ],
        scratch_types=[
            pltpu.VMEM_SHARED((N, D), jnp.float32),
            pltpu.VMEM_SHARED((CR, D), jnp.float32),
            pltpu.VMEM((CH,), jnp.int32),
            pltpu.VMEM((CH,), jnp.int32),
            pltpu.VMEM((CH,), jnp.int32),
            pltpu.VMEM((CH,), jnp.int32),
            pltpu.VMEM((CH, D), jnp.float32),
            pltpu.VMEM((CH, D), jnp.float32),
            pltpu.VMEM((L * npt,), jnp.float32),
            pltpu.VMEM((DCH,), jnp.int32),
            pltpu.VMEM((npt // D, D), jnp.float32),
            pltpu.SemaphoreType.DMA,
            pltpu.SemaphoreType.DMA,
            pltpu.SemaphoreType.DMA,
            pltpu.SemaphoreType.DMA,
            pltpu.SemaphoreType.DMA,
            pltpu.SemaphoreType.DMA,
        ],
    )
    def sc_kernel(x_hbm, src_hbm, dst_hbm, agg_out, cnt_out,
                  acc, cgrid, sidx0, didx0, sidx1, didx1, rowbuf0, rowbuf1,
                  hist, dchunk, ctot,
                  semg0, semg1, semi0s, semi0d, semi1s, semi1d):
        c = lax.axis_index("c")
        s = lax.axis_index("s")
        wid = s * NC + c
        iota16 = lax.iota(jnp.int32, 16)
        ones16 = jnp.full((L,), 1.0, jnp.float32)

        zbuf = rowbuf0
        _fill_2d(zbuf, zrows, 0.0)
        _fill_1d(hist, L * npt, 0.0)

        row0 = s * rpt

        def zero_rows(tgt, start, count):
            done = 0
            while done < count:
                m = min(zrows, count - done)
                pltpu.sync_copy(zbuf.at[pl.ds(0, m)],
                                tgt.at[pl.ds(start + done, m)])
                done += m

        zero_rows(acc, row0, rpt)
        if rextra:
            @pl.when(s == NS - 1)
            def _():
                zero_rows(acc, NS * rpt, rextra)
        plsc.subcore_barrier()

        ebase = wid * ept
        emax = E - CH

        def ebase_of(i):
            return jnp.minimum(ebase + i * CH, emax)

        def start_idx(i, si, di, sis, sid):
            b = ebase_of(i)
            pltpu.async_copy(src_hbm.at[pl.ds(b, CH)], si, sis)
            pltpu.async_copy(dst_hbm.at[pl.ds(b, CH)], di, sid)

        def wait_idx(i, si, di, sis, sid):
            b = ebase_of(i)
            pltpu.make_async_copy(src_hbm.at[pl.ds(b, CH)], si, sis).wait()
            pltpu.make_async_copy(dst_hbm.at[pl.ds(b, CH)], di, sid).wait()

        def wait_gather(si, rb, sg):
            pltpu.make_async_copy(x_hbm.at[si], rb, sg).wait()

        assert nfull % 2 == 1
        ngrp = nfull // 2

        pltpu.sync_copy(src_hbm.at[pl.ds(ebase, CH)], sidx0)
        pltpu.sync_copy(dst_hbm.at[pl.ds(ebase, CH)], didx0)
        pltpu.async_copy(x_hbm.at[sidx0], rowbuf0, semg0)
        start_idx(1, sidx1, didx1, semi1s, semi1d)

        def pair(g, _):
            i0 = 2 * g
            i1 = i0 + 1
            wait_idx(i1, sidx1, didx1, semi1s, semi1d)
            pltpu.async_copy(x_hbm.at[sidx1], rowbuf1, semg1)
            wait_gather(sidx0, rowbuf0, semg0)
            pltpu.sync_copy(rowbuf0, acc.at[didx0], add=True)
            start_idx(i0 + 2, sidx0, didx0, semi0s, semi0d)
            wait_gather(sidx1, rowbuf1, semg1)
            pltpu.sync_copy(rowbuf1, acc.at[didx1], add=True)
            start_idx(i1 + 2, sidx1, didx1, semi1s, semi1d)
            wait_idx(i0 + 2, sidx0, didx0, semi0s, semi0d)
            pltpu.async_copy(x_hbm.at[sidx0], rowbuf0, semg0)
            return 0

        lax.fori_loop(0, ngrp, pair, 0)

        wait_idx(nfull, sidx1, didx1, semi1s, semi1d)
        wait_gather(sidx0, rowbuf0, semg0)
        pltpu.sync_copy(rowbuf0, acc.at[didx0], add=True)

        nbase = s * npt
        ecore = ept * NS

        def cchunk(q, _):
            e0 = q * DCH
            blk = e0 // ept
            base = (blk * NC + c) * ept + (e0 - blk * ept)
            pltpu.sync_copy(dst_hbm.at[pl.ds(base, DCH)], dchunk)

            def body(k, _):
                dv = dchunk[pl.ds(k * L, L)] - nbase
                msk = (dv >= 0) & (dv < npt)
                plsc.addupdate_scatter(hist, [iota16 * npt + dv], ones16, mask=msk)
                return 0

            lax.fori_loop(0, DCH // L, body, 0)
            return 0

        lax.fori_loop(0, ecore // DCH, cchunk, 0)

        for j in range(npt // L):
            tot = hist[pl.ds(j * L, L)]
            for r in range(1, L):
                tot = tot + hist[pl.ds(r * npt + j * L, L)]
            ctot[(j * L) // D, pl.ds((j * L) % D, L)] = tot
        pltpu.sync_copy(ctot, cgrid.at[pl.ds(s * (npt // D), npt // D)])

        plsc.subcore_barrier()

        pltpu.sync_copy(acc.at[pl.ds(row0, rpt)],
                        agg_out.at[c, pl.ds(row0, rpt)])
        if rextra:
            @pl.when(s == NS - 1)
            def _():
                pltpu.sync_copy(acc.at[pl.ds(NS * rpt, rextra)],
                                agg_out.at[c, pl.ds(NS * rpt, rextra)])

        @pl.when(s == 0)
        def _():
            pltpu.sync_copy(cgrid, cnt_out.at[c])

    return sc_kernel(x, src, dst)


def _tc_body(x_ref, wl_ref, wr_ref, bl_ref, agg_ref, cnt_ref, out_ref):
    a = agg_ref[0] + agg_ref[1]
    cvec = cnt_ref[0] + cnt_ref[1]
    mean = a / jnp.maximum(cvec, 1.0)
    h = (jnp.dot(mean, wl_ref[...], preferred_element_type=jnp.float32)
         + jnp.dot(x_ref[...], wr_ref[...], preferred_element_type=jnp.float32)
         + bl_ref[...])
    h = jnp.where(h >= 0, h, NEG_SLOPE * h)
    nrm = jnp.sqrt(jnp.sum(h * h, axis=-1, keepdims=True))
    out_ref[...] = h / jnp.maximum(nrm, 1e-12)


def _tc_post(x, W_l, W_r, b_l2, agg, cnt):
    N, D = x.shape
    H = W_l.shape[1]
    R = 1000
    assert N % R == 0
    grid = (N // R,)
    return pl.pallas_call(
        _tc_body,
        grid=grid,
        in_specs=[
            pl.BlockSpec((R, D), lambda i: (i, 0)),
            pl.BlockSpec((D, H), lambda i: (0, 0)),
            pl.BlockSpec((D, H), lambda i: (0, 0)),
            pl.BlockSpec((1, H), lambda i: (0, 0)),
            pl.BlockSpec((NC, R, D), lambda i: (0, i, 0)),
            pl.BlockSpec((NC, R, 1), lambda i: (0, i, 0)),
        ],
        out_specs=pl.BlockSpec((R, H), lambda i: (i, 0)),
        out_shape=jax.ShapeDtypeStruct((N, H), jnp.float32),
    )(x, W_l, W_r, b_l2, agg, cnt)


def kernel(x, edge_index, W_l, W_r, b_l):
    src = edge_index[0]
    dst = edge_index[1]
    agg, cnt_grid = _sc_aggregate(x, src, dst)
    N = x.shape[0]
    cnt = cnt_grid.reshape(NC, -1)[:, :N, None]
    return _tc_post(x, W_l, W_r, b_l.reshape(1, -1), agg, cnt)


# speedup vs baseline: 7.8153x; 1.1508x over previous
import functools

import jax
import jax.numpy as jnp
from jax import lax
from jax.experimental import pallas as pl
from jax.experimental.pallas import tpu as pltpu, tpu_sc as plsc

NEG_SLOPE = 0.01

NC = 2
NS = 16
L = 16

CH = 80


def _fill_1d(ref, n, value):
    def body(i, _):
        ref[pl.ds(i * L, L)] = jnp.full((L,), value, jnp.float32)
        return 0

    lax.fori_loop(0, n // L, body, 0)


def _fill_2d(ref, rows, value):
    cols = ref.shape[1]

    def body(r, _):
        for j in range(cols // L):
            ref[r, pl.ds(j * L, L)] = jnp.full((L,), value, jnp.float32)
        return 0

    lax.fori_loop(0, rows, body, 0)


def _sc_aggregate(x, src, dst):
    N, D = x.shape
    E = src.shape[0]
    NW = NC * NS
    assert E % NW == 0
    ept = E // NW
    nfull = ept // CH
    assert ept % CH == 0 and CH % 8 == 0
    rpt = (N // NS) & ~7
    rextra = N - NS * rpt
    assert rextra % 8 == 0 and rextra <= rpt
    zrows = CH
    npt = -(-N // (NS * D)) * D
    CR = NS * npt // D
    DCH = 2000
    assert (ept * NS) % DCH == 0 and ept % DCH == 0 and DCH % L == 0
    assert npt % D == 0 and CR % 8 == 0 and NS * npt >= N

    mesh = plsc.VectorSubcoreMesh(core_axis_name="c", subcore_axis_name="s")

    @functools.partial(
        pl.kernel,
        mesh=mesh,
        compiler_params=pltpu.CompilerParams(needs_layout_passes=False),
        out_type=[
            jax.ShapeDtypeStruct((NC, N, D), jnp.float32),
            jax.ShapeDtypeStruct((NC, CR, D), jnp.float32),
        ],
        scratch_types=[
            pltpu.VMEM_SHARED((N, D), jnp.float32),
            pltpu.VMEM_SHARED((CR, D), jnp.float32),
            pltpu.VMEM((CH,), jnp.int32),
            pltpu.VMEM((CH,), jnp.int32),
            pltpu.VMEM((CH,), jnp.int32),
            pltpu.VMEM((CH,), jnp.int32),
            pltpu.VMEM((CH, D), jnp.float32),
            pltpu.VMEM((CH, D), jnp.float32),
            pltpu.VMEM((L * npt,), jnp.float32),
            pltpu.VMEM((DCH,), jnp.int32),
            pltpu.VMEM((DCH,), jnp.int32),
            pltpu.VMEM((npt // D, D), jnp.float32),
            pltpu.SemaphoreType.DMA,
            pltpu.SemaphoreType.DMA,
            pltpu.SemaphoreType.DMA,
            pltpu.SemaphoreType.DMA,
            pltpu.SemaphoreType.DMA,
            pltpu.SemaphoreType.DMA,
            pltpu.SemaphoreType.DMA,
            pltpu.SemaphoreType.DMA,
        ],
    )
    def sc_kernel(x_hbm, src_hbm, dst_hbm, agg_out, cnt_out,
                  acc, cgrid, sidx0, didx0, sidx1, didx1, rowbuf0, rowbuf1,
                  hist, dchunk0, dchunk1, ctot,
                  semg0, semg1, semi0s, semi0d, semi1s, semi1d,
                  semd0, semd1):
        c = lax.axis_index("c")
        s = lax.axis_index("s")
        wid = s * NC + c
        iota16 = lax.iota(jnp.int32, 16)
        ones16 = jnp.full((L,), 1.0, jnp.float32)

        zbuf = rowbuf0
        _fill_2d(zbuf, zrows, 0.0)
        _fill_1d(hist, L * npt, 0.0)

        row0 = s * rpt

        def zero_rows(tgt, start, count):
            done = 0
            while done < count:
                m = min(zrows, count - done)
                pltpu.sync_copy(zbuf.at[pl.ds(0, m)],
                                tgt.at[pl.ds(start + done, m)])
                done += m

        zero_rows(acc, row0, rpt)
        if rextra:
            @pl.when(s == NS - 1)
            def _():
                zero_rows(acc, NS * rpt, rextra)
        plsc.subcore_barrier()

        ebase = wid * ept
        emax = E - CH

        def ebase_of(i):
            return jnp.minimum(ebase + i * CH, emax)

        def start_idx(i, si, di, sis, sid):
            b = ebase_of(i)
            pltpu.async_copy(src_hbm.at[pl.ds(b, CH)], si, sis)
            pltpu.async_copy(dst_hbm.at[pl.ds(b, CH)], di, sid)

        def wait_idx(i, si, di, sis, sid):
            b = ebase_of(i)
            pltpu.make_async_copy(src_hbm.at[pl.ds(b, CH)], si, sis).wait()
            pltpu.make_async_copy(dst_hbm.at[pl.ds(b, CH)], di, sid).wait()

        def wait_gather(si, rb, sg):
            pltpu.make_async_copy(x_hbm.at[si], rb, sg).wait()

        assert nfull % 2 == 1
        ngrp = nfull // 2

        pltpu.sync_copy(src_hbm.at[pl.ds(ebase, CH)], sidx0)
        pltpu.sync_copy(dst_hbm.at[pl.ds(ebase, CH)], didx0)
        pltpu.async_copy(x_hbm.at[sidx0], rowbuf0, semg0)
        start_idx(1, sidx1, didx1, semi1s, semi1d)

        def pair(g, _):
            i0 = 2 * g
            i1 = i0 + 1
            wait_idx(i1, sidx1, didx1, semi1s, semi1d)
            pltpu.async_copy(x_hbm.at[sidx1], rowbuf1, semg1)
            wait_gather(sidx0, rowbuf0, semg0)
            pltpu.sync_copy(rowbuf0, acc.at[didx0], add=True)
            start_idx(i0 + 2, sidx0, didx0, semi0s, semi0d)
            wait_gather(sidx1, rowbuf1, semg1)
            pltpu.sync_copy(rowbuf1, acc.at[didx1], add=True)
            start_idx(i1 + 2, sidx1, didx1, semi1s, semi1d)
            wait_idx(i0 + 2, sidx0, didx0, semi0s, semi0d)
            pltpu.async_copy(x_hbm.at[sidx0], rowbuf0, semg0)
            return 0

        lax.fori_loop(0, ngrp, pair, 0)

        wait_idx(nfull, sidx1, didx1, semi1s, semi1d)
        wait_gather(sidx0, rowbuf0, semg0)
        pltpu.sync_copy(rowbuf0, acc.at[didx0], add=True)

        nbase = s * npt
        ecore = ept * NS
        ncq = ecore // DCH
        assert ncq % 2 == 0

        def cbase(q):
            qc = jnp.minimum(q, ncq - 1)
            e0 = qc * DCH
            blk = e0 // ept
            return (blk * NC + c) * ept + (e0 - blk * ept)

        def cstart(q, buf, sem):
            pltpu.async_copy(dst_hbm.at[pl.ds(cbase(q), DCH)], buf, sem)

        def cwait(q, buf, sem):
            pltpu.make_async_copy(
                dst_hbm.at[pl.ds(cbase(q), DCH)], buf, sem).wait()

        def cproc(buf):
            def body(k, _):
                dv = buf[pl.ds(k * L, L)] - nbase
                msk = (dv >= 0) & (dv < npt)
                plsc.addupdate_scatter(
                    hist, [iota16 * npt + dv], ones16, mask=msk)
                return 0

            lax.fori_loop(0, DCH // L, body, 0)

        cstart(0, dchunk0, semd0)
        cstart(1, dchunk1, semd1)

        def cpair(t, _):
            q0 = 2 * t
            cwait(q0, dchunk0, semd0)
            cproc(dchunk0)
            cstart(q0 + 2, dchunk0, semd0)
            cwait(q0 + 1, dchunk1, semd1)
            cproc(dchunk1)
            cstart(q0 + 3, dchunk1, semd1)
            return 0

        lax.fori_loop(0, ncq // 2, cpair, 0)
        cwait(ncq, dchunk0, semd0)
        cwait(ncq + 1, dchunk1, semd1)

        for j in range(npt // L):
            tot = hist[pl.ds(j * L, L)]
            for r in range(1, L):
                tot = tot + hist[pl.ds(r * npt + j * L, L)]
            ctot[(j * L) // D, pl.ds((j * L) % D, L)] = tot
        pltpu.sync_copy(ctot, cgrid.at[pl.ds(s * (npt // D), npt // D)])

        plsc.subcore_barrier()

        pltpu.sync_copy(acc.at[pl.ds(row0, rpt)],
                        agg_out.at[c, pl.ds(row0, rpt)])
        if rextra:
            @pl.when(s == NS - 1)
            def _():
                pltpu.sync_copy(acc.at[pl.ds(NS * rpt, rextra)],
                                agg_out.at[c, pl.ds(NS * rpt, rextra)])

        @pl.when(s == 0)
        def _():
            pltpu.sync_copy(cgrid, cnt_out.at[c])

    return sc_kernel(x, src, dst)


def _tc_body(x_ref, wl_ref, wr_ref, bl_ref, agg_ref, cnt_ref, out_ref):
    a = agg_ref[0] + agg_ref[1]
    cvec = cnt_ref[0] + cnt_ref[1]
    mean = a / jnp.maximum(cvec, 1.0)
    h = (jnp.dot(mean, wl_ref[...], preferred_element_type=jnp.float32)
         + jnp.dot(x_ref[...], wr_ref[...], preferred_element_type=jnp.float32)
         + bl_ref[...])
    h = jnp.where(h >= 0, h, NEG_SLOPE * h)
    nrm = jnp.sqrt(jnp.sum(h * h, axis=-1, keepdims=True))
    out_ref[...] = h / jnp.maximum(nrm, 1e-12)


def _tc_post(x, W_l, W_r, b_l2, agg, cnt):
    N, D = x.shape
    H = W_l.shape[1]
    R = 1000
    assert N % R == 0
    grid = (N // R,)
    return pl.pallas_call(
        _tc_body,
        grid=grid,
        in_specs=[
            pl.BlockSpec((R, D), lambda i: (i, 0)),
            pl.BlockSpec((D, H), lambda i: (0, 0)),
            pl.BlockSpec((D, H), lambda i: (0, 0)),
            pl.BlockSpec((1, H), lambda i: (0, 0)),
            pl.BlockSpec((NC, R, D), lambda i: (0, i, 0)),
            pl.BlockSpec((NC, R, 1), lambda i: (0, i, 0)),
        ],
        out_specs=pl.BlockSpec((R, H), lambda i: (i, 0)),
        out_shape=jax.ShapeDtypeStruct((N, H), jnp.float32),
    )(x, W_l, W_r, b_l2, agg, cnt)


def kernel(x, edge_index, W_l, W_r, b_l):
    src = edge_index[0]
    dst = edge_index[1]
    agg, cnt_grid = _sc_aggregate(x, src, dst)
    N = x.shape[0]
    cnt = cnt_grid.reshape(NC, -1)[:, :N, None]
    return _tc_post(x, W_l, W_r, b_l.reshape(1, -1), agg, cnt)
